# baseline (speedup 1.0000x reference)
import functools

import jax
import jax.numpy as jnp
from jax import lax
from jax.experimental import pallas as pl
from jax.experimental.pallas import tpu as pltpu
from jax.experimental.pallas import tpu_sc as plsc


def _plane_body(row_ref, col_ref, o_ref):
    d, h = row_ref.shape
    w = col_ref.shape[1]
    hw = h * w
    p_i = jax.lax.broadcasted_iota(jnp.int32, (h, hw), 1) // w
    p_j = jax.lax.broadcasted_iota(jnp.int32, (w, hw), 1) % w
    ii = jax.lax.broadcasted_iota(jnp.int32, (h, hw), 0)
    jj = jax.lax.broadcasted_iota(jnp.int32, (w, hw), 0)
    R = (p_i == ii).astype(jnp.float32)
    C = (p_j == jj).astype(jnp.float32)
    o_ref[...] = (
        jnp.dot(row_ref[...], R, preferred_element_type=jnp.float32,
                precision=jax.lax.Precision.HIGHEST)
        + jnp.dot(col_ref[...], C, preferred_element_type=jnp.float32,
                  precision=jax.lax.Precision.HIGHEST)
    )


def kernel(x, mask, row_embed, col_embed):
    B = x.shape[0]
    h, w = x.shape[-2], x.shape[-1]
    hw = h * w
    d = row_embed.shape[-1]
    rowT = row_embed.T
    colT = col_embed.T

    plane = pl.pallas_call(
        _plane_body,
        in_specs=[
            pl.BlockSpec((d, h), lambda: (0, 0)),
            pl.BlockSpec((d, w), lambda: (0, 0)),
        ],
        out_specs=pl.BlockSpec((d, hw), lambda: (0, 0)),
        out_shape=jax.ShapeDtypeStruct((d, hw), jnp.float32),
    )(rowT, colT)

    info = plsc.get_sparse_core_info()
    NC, NS = info.num_cores, info.num_subcores
    NW = NC * NS
    C = max(1, NW // B)
    while d % C != 0 or (B * C) % NW != 0:
        C -= 1
    dk = d // C
    per_w = (B * C) // NW

    mesh = plsc.VectorSubcoreMesh(core_axis_name="c", subcore_axis_name="s")

    @functools.partial(
        pl.kernel,
        mesh=mesh,
        out_type=jax.ShapeDtypeStruct((B, d, hw), jnp.float32),
    )
    def _fanout(plane_hbm, out_hbm):
        wid = lax.axis_index("s") * NC + lax.axis_index("c")
        for k in range(per_w):
            item = wid * per_w + k
            b = item // C
            c0 = (item % C) * dk
            pltpu.sync_copy(
                plane_hbm.at[pl.ds(c0, dk)],
                out_hbm.at[b].at[pl.ds(c0, dk)],
            )

    out = _fanout(plane)
    return out.reshape(B, d, h, w)


# speedup vs baseline: 19.0086x; 19.0086x over previous
import jax
import jax.numpy as jnp
from jax.experimental import pallas as pl
from jax.experimental.pallas import tpu as pltpu

_NSRC = 4


def _body(row_ref, col_ref, o_ref, s_ref, sems):
    d, h = row_ref.shape
    w = col_ref.shape[1]
    hw = h * w
    B = o_ref.shape[0]

    p_i = jax.lax.broadcasted_iota(jnp.int32, (h, hw), 1) // w
    p_j = jax.lax.broadcasted_iota(jnp.int32, (w, hw), 1) % w
    ii = jax.lax.broadcasted_iota(jnp.int32, (h, hw), 0)
    jj = jax.lax.broadcasted_iota(jnp.int32, (w, hw), 0)
    R = (p_i == ii).astype(jnp.float32)
    C = (p_j == jj).astype(jnp.float32)
    s_ref[0] = (
        jnp.dot(row_ref[...], R, preferred_element_type=jnp.float32,
                precision=jax.lax.Precision.HIGHEST)
        + jnp.dot(col_ref[...], C, preferred_element_type=jnp.float32,
                  precision=jax.lax.Precision.HIGHEST)
    )
    reps = [pltpu.make_async_copy(s_ref.at[0], s_ref.at[k], sems.at[k])
            for k in range(1, _NSRC)]
    for r in reps:
        r.start()
    for r in reps:
        r.wait()

    copies = [
        pltpu.make_async_copy(s_ref.at[b % _NSRC], o_ref.at[b], sems.at[b])
        for b in range(B)
    ]
    for b, c in enumerate(copies):
        c.start(priority=b % 2)
    for c in copies:
        c.wait()


def kernel(x, mask, row_embed, col_embed):
    B = x.shape[0]
    h, w = x.shape[-2], x.shape[-1]
    d = row_embed.shape[-1]
    rowT = row_embed.T
    colT = col_embed.T
    out = pl.pallas_call(
        _body,
        in_specs=[
            pl.BlockSpec((d, h), lambda: (0, 0)),
            pl.BlockSpec((d, w), lambda: (0, 0)),
        ],
        out_specs=pl.BlockSpec(memory_space=pl.ANY),
        out_shape=jax.ShapeDtypeStruct((B, d, h * w), jnp.float32),
        scratch_shapes=[
            pltpu.VMEM((_NSRC, d, h * w), jnp.float32),
            pltpu.SemaphoreType.DMA((max(B, _NSRC),)),
        ],
    )(rowT, colT)
    return out.reshape(B, d, h, w)
